# baseline (speedup 1.0000x reference)
import functools

import jax
import jax.numpy as jnp
from jax import lax
from jax.experimental import pallas as pl
from jax.experimental.pallas import tpu as pltpu
from jax.experimental.pallas import tpu_sc as plsc

_B = 64
_T = 1000
_BB = 8
_L = 16


def _sc_gather_body(sac_hbm, somac_hbm, t_hbm, c1_hbm, c2_hbm,
                    t_v, c1_v, c2_v, sem1, sem2):
    wid = lax.axis_index("s") * 2 + lax.axis_index("c")

    @pl.when(wid == 0)
    def _():
        pltpu.sync_copy(t_hbm, t_v)
        cp1 = pltpu.async_copy(sac_hbm.at[t_v], c1_v, sem1)
        cp2 = pltpu.async_copy(somac_hbm.at[t_v], c2_v, sem2)
        cp1.wait()
        cp2.wait()
        pltpu.sync_copy(c1_v, c1_hbm)
        pltpu.sync_copy(c2_v, c2_hbm)


@jax.jit
def _sc_gather(sac, somac, t):
    f32 = jnp.float32
    return pl.kernel(
        _sc_gather_body,
        mesh=plsc.VectorSubcoreMesh(core_axis_name="c", subcore_axis_name="s"),
        out_type=[jax.ShapeDtypeStruct((_B,), f32),
                  jax.ShapeDtypeStruct((_B,), f32)],
        scratch_types=[
            pltpu.VMEM((_B,), jnp.int32),
            pltpu.VMEM((_B,), f32),
            pltpu.VMEM((_B,), f32),
            pltpu.SemaphoreType.DMA,
            pltpu.SemaphoreType.DMA,
        ],
    )(sac, somac, t)


def _fma_body(c1_ref, c2_ref, x_ref, n_ref, o_ref):
    i = pl.program_id(0)
    for r in range(_BB):
        c1 = c1_ref[i * _BB + r]
        c2 = c2_ref[i * _BB + r]
        o_ref[r] = c1 * x_ref[r] + c2 * n_ref[r]


@jax.jit
def _tc_fma(c1, c2, x, n):
    blk = (_BB,) + x.shape[1:]
    imap = lambda i: (i, 0, 0, 0)
    smem_spec = pl.BlockSpec(memory_space=pltpu.SMEM)
    return pl.pallas_call(
        _fma_body,
        grid=(_B // _BB,),
        in_specs=[
            smem_spec,
            smem_spec,
            pl.BlockSpec(blk, imap),
            pl.BlockSpec(blk, imap),
        ],
        out_specs=pl.BlockSpec(blk, imap),
        out_shape=jax.ShapeDtypeStruct(x.shape, jnp.float32),
    )(c1, c2, x, n)


def kernel(x_start, noise, sqrt_alphas_cumprod, sqrt_one_minus_alphas_cumprod, t):
    c1, c2 = _sc_gather(sqrt_alphas_cumprod, sqrt_one_minus_alphas_cumprod, t)
    return _tc_fma(c1, c2, x_start, noise)


# speedup vs baseline: 2.6264x; 2.6264x over previous
import functools

import jax
import jax.numpy as jnp
from jax.experimental import pallas as pl
from jax.experimental.pallas import tpu as pltpu

_B = 64
_BB = 8


def _fused_body(t_ref, sac_ref, somac_ref, x_ref, n_ref, o_ref):
    i = pl.program_id(0)
    for r in range(_BB):
        ti = t_ref[i * _BB + r]
        c1 = sac_ref[ti]
        c2 = somac_ref[ti]
        o_ref[r] = c1 * x_ref[r] + c2 * n_ref[r]


@jax.jit
def _tc_fused(t, sac, somac, x, n):
    blk = (_BB,) + x.shape[1:]
    imap = lambda i, *_: (i, 0, 0, 0)
    grid_spec = pltpu.PrefetchScalarGridSpec(
        num_scalar_prefetch=3,
        grid=(_B // _BB,),
        in_specs=[
            pl.BlockSpec(blk, imap),
            pl.BlockSpec(blk, imap),
        ],
        out_specs=pl.BlockSpec(blk, imap),
    )
    return pl.pallas_call(
        _fused_body,
        grid_spec=grid_spec,
        out_shape=jax.ShapeDtypeStruct(x.shape, jnp.float32),
        compiler_params=pltpu.CompilerParams(
            dimension_semantics=("parallel",)),
    )(t, sac, somac, x, n)


def kernel(x_start, noise, sqrt_alphas_cumprod, sqrt_one_minus_alphas_cumprod, t):
    return _tc_fused(t, sqrt_alphas_cumprod, sqrt_one_minus_alphas_cumprod,
                     x_start, noise)


# speedup vs baseline: 2.9995x; 1.1420x over previous
import functools

import jax
import jax.numpy as jnp
from jax.experimental import pallas as pl
from jax.experimental.pallas import tpu as pltpu

_B = 64
_BB = 16


def _fused_body(t_ref, sac_ref, somac_ref, x_ref, n_ref, o_ref):
    i = pl.program_id(0)
    for r in range(_BB):
        ti = t_ref[i * _BB + r]
        c1 = sac_ref[ti]
        c2 = somac_ref[ti]
        o_ref[r] = c1 * x_ref[r] + c2 * n_ref[r]


@jax.jit
def _tc_fused(t, sac, somac, x, n):
    blk = (_BB,) + x.shape[1:]
    imap = lambda i, *_: (i, 0, 0, 0)
    grid_spec = pltpu.PrefetchScalarGridSpec(
        num_scalar_prefetch=3,
        grid=(_B // _BB,),
        in_specs=[
            pl.BlockSpec(blk, imap),
            pl.BlockSpec(blk, imap),
        ],
        out_specs=pl.BlockSpec(blk, imap),
    )
    return pl.pallas_call(
        _fused_body,
        grid_spec=grid_spec,
        out_shape=jax.ShapeDtypeStruct(x.shape, jnp.float32),
        compiler_params=pltpu.CompilerParams(
            dimension_semantics=("parallel",)),
    )(t, sac, somac, x, n)


def kernel(x_start, noise, sqrt_alphas_cumprod, sqrt_one_minus_alphas_cumprod, t):
    return _tc_fused(t, sqrt_alphas_cumprod, sqrt_one_minus_alphas_cumprod,
                     x_start, noise)


# speedup vs baseline: 3.0614x; 1.0206x over previous
import functools

import jax
import jax.numpy as jnp
from jax.experimental import pallas as pl
from jax.experimental.pallas import tpu as pltpu

_B = 64
_BB = 32


def _fused_body(t_ref, sac_ref, somac_ref, x_ref, n_ref, o_ref):
    i = pl.program_id(0)
    for r in range(_BB):
        ti = t_ref[i * _BB + r]
        c1 = sac_ref[ti]
        c2 = somac_ref[ti]
        o_ref[r] = c1 * x_ref[r] + c2 * n_ref[r]


@jax.jit
def _tc_fused(t, sac, somac, x, n):
    blk = (_BB,) + x.shape[1:]
    imap = lambda i, *_: (i, 0, 0, 0)
    grid_spec = pltpu.PrefetchScalarGridSpec(
        num_scalar_prefetch=3,
        grid=(_B // _BB,),
        in_specs=[
            pl.BlockSpec(blk, imap),
            pl.BlockSpec(blk, imap),
        ],
        out_specs=pl.BlockSpec(blk, imap),
    )
    return pl.pallas_call(
        _fused_body,
        grid_spec=grid_spec,
        out_shape=jax.ShapeDtypeStruct(x.shape, jnp.float32),
        compiler_params=pltpu.CompilerParams(
            dimension_semantics=("parallel",)),
    )(t, sac, somac, x, n)


def kernel(x_start, noise, sqrt_alphas_cumprod, sqrt_one_minus_alphas_cumprod, t):
    return _tc_fused(t, sqrt_alphas_cumprod, sqrt_one_minus_alphas_cumprod,
                     x_start, noise)
